# baseline (speedup 1.0000x reference)
import functools

import jax
import jax.numpy as jnp
from jax import lax
from jax.experimental import pallas as pl
from jax.experimental.pallas import tpu as pltpu
from jax.experimental.pallas import tpu_sc as plsc

_NC = 2
_NS = 16
_CHUNK = 128
_IB = 16
_LANES = 16


def _sc_mesh():
    return plsc.VectorSubcoreMesh(core_axis_name="c", subcore_axis_name="s")


def _sc_degree(dst2, zeros_flat, np_):
    nchunk = dst2.shape[0]
    nw = _NC * _NS
    niter = nchunk // nw
    npw = np_ // _NS

    @functools.partial(
        pl.kernel,
        out_type=jax.ShapeDtypeStruct((_NC, np_), jnp.float32),
        mesh=_sc_mesh(),
        compiler_params=pltpu.CompilerParams(needs_layout_passes=False),
        scratch_types=[
            pltpu.VMEM((niter, _CHUNK), jnp.int32),
            pltpu.VMEM((np_,), jnp.float32),
            pltpu.VMEM((_NS, npw), jnp.float32),
            pltpu.VMEM_SHARED((_NS, np_), jnp.float32),
        ],
    )
    def k(dst_hbm, z_hbm, out_hbm, idx_all, hist_v, red_v, stage_sh):
        c = lax.axis_index("c")
        s = lax.axis_index("s")
        w = c * _NS + s
        pltpu.sync_copy(z_hbm, hist_v)
        pltpu.sync_copy(dst_hbm.at[pl.ds(w * niter, niter)], idx_all)
        ones = jnp.full((_LANES,), 1.0, jnp.float32)

        def chunk_body(i, carry):
            for j in range(_CHUNK // _LANES):
                dvec = idx_all[i, pl.ds(j * _LANES, _LANES)]
                plsc.addupdate_scatter(hist_v, (dvec,), ones)
            return carry

        lax.fori_loop(0, niter, chunk_body, 0)

        pltpu.sync_copy(hist_v, stage_sh.at[s])
        plsc.subcore_barrier()
        for r in range(_NS):
            pltpu.sync_copy(stage_sh.at[r, pl.ds(s * npw, npw)], red_v.at[r])

        def red_body(j, carry):
            acc = red_v[0, pl.ds(j * _LANES, _LANES)]
            for r in range(1, _NS):
                acc = acc + red_v[r, pl.ds(j * _LANES, _LANES)]
            hist_v[pl.ds(j * _LANES, _LANES)] = acc
            return carry

        lax.fori_loop(0, npw // _LANES, red_body, 0)
        pltpu.sync_copy(hist_v.at[pl.ds(0, npw)],
                        out_hbm.at[c, pl.ds(s * npw, npw)])

    return k(dst2, zeros_flat)


def _sc_scatter(h, src2, dst2, zeros_slab, np_):
    nchunk = src2.shape[0]
    dh = h.shape[1]
    nw = _NC * _NS
    niter = nchunk // nw
    rows_per = np_ // _NS
    ib = _IB

    @functools.partial(
        pl.kernel,
        out_type=jax.ShapeDtypeStruct((_NC, np_, dh), jnp.float32),
        mesh=_sc_mesh(),
        compiler_params=pltpu.CompilerParams(needs_layout_passes=False),
        scratch_types=[
            pltpu.VMEM((ib, _CHUNK), jnp.int32),
            pltpu.VMEM((ib, _CHUNK), jnp.int32),
            pltpu.VMEM((_CHUNK, dh), jnp.float32),
            pltpu.VMEM((_CHUNK, dh), jnp.float32),
            pltpu.VMEM_SHARED((np_, dh), jnp.float32),
            pltpu.SemaphoreType.DMA,
            pltpu.SemaphoreType.DMA,
        ],
    )
    def k(h_hbm, src_hbm, dst_hbm, z_hbm, out_hbm, isrc, idst, rows0, rows1,
          acc_sh, sem0, sem1):
        c = lax.axis_index("c")
        s = lax.axis_index("s")
        w = c * _NS + s
        pltpu.sync_copy(z_hbm, acc_sh.at[pl.ds(s * rows_per, rows_per)])
        plsc.subcore_barrier()

        def outer(o, carry):
            base = w * niter + o * ib
            pltpu.sync_copy(src_hbm.at[pl.ds(base, ib)], isrc)
            pltpu.sync_copy(dst_hbm.at[pl.ds(base, ib)], idst)
            def body(i, carry2):
                pltpu.async_copy(h_hbm.at[isrc.at[2 * i]], rows0, sem0).wait()
                pltpu.sync_copy(rows0, acc_sh.at[idst.at[2 * i]], add=True)
                pltpu.async_copy(h_hbm.at[isrc.at[2 * i + 1]], rows1,
                                 sem1).wait()
                pltpu.sync_copy(rows1, acc_sh.at[idst.at[2 * i + 1]],
                                add=True)
                return carry2

            lax.fori_loop(0, ib // 2, body, 0)
            return carry

        lax.fori_loop(0, niter // ib, outer, 0)
        plsc.subcore_barrier()
        pltpu.sync_copy(acc_sh.at[pl.ds(s * rows_per, rows_per)],
                        out_hbm.at[c, pl.ds(s * rows_per, rows_per)])

    return k(h, src2, dst2, zeros_slab)


def _tc_layer1(x_pad, W1, deg3, r_blk):
    np_, d_in = x_pad.shape
    d_h = W1.shape[1]
    half = d_h // 2
    grid = np_ // r_blk

    def body(x_ref, w_ref, deg_ref, lo_ref, hi_ref, dis_ref):
        deg = deg_ref[...]
        dis = lax.rsqrt(deg[0] + deg[1] + 1.0)
        h = jnp.dot(x_ref[...], w_ref[...], preferred_element_type=jnp.float32)
        lo_ref[...] = h[:, :half] * dis
        hi_ref[...] = h[:, half:] * dis
        dis_ref[...] = dis

    return pl.pallas_call(
        body,
        grid=(grid,),
        in_specs=[
            pl.BlockSpec((r_blk, d_in), lambda i: (i, 0)),
            pl.BlockSpec((d_in, d_h), lambda i: (0, 0)),
            pl.BlockSpec((_NC, r_blk, 1), lambda i: (0, i, 0)),
        ],
        out_specs=[
            pl.BlockSpec((r_blk, half), lambda i: (i, 0)),
            pl.BlockSpec((r_blk, half), lambda i: (i, 0)),
            pl.BlockSpec((r_blk, 1), lambda i: (i, 0)),
        ],
        out_shape=[
            jax.ShapeDtypeStruct((np_, half), jnp.float32),
            jax.ShapeDtypeStruct((np_, half), jnp.float32),
            jax.ShapeDtypeStruct((np_, 1), jnp.float32),
        ],
    )(x_pad, W1, deg3)


def _tc_layer2(acc_lo, acc_hi, h_lo, h_hi, dis, b1, W2, r_blk):
    np_ = h_lo.shape[0]
    half = h_lo.shape[1]
    d_out = W2.shape[1]
    grid = np_ // r_blk

    def body(alo_ref, ahi_ref, hlo_ref, hhi_ref, dis_ref, b1_ref, w2_ref,
             out_ref):
        dis_v = dis_ref[...]
        alo = alo_ref[...]
        ahi = ahi_ref[...]
        b1v = b1_ref[...]
        zlo = jnp.maximum((alo[0] + alo[1] + hlo_ref[...]) * dis_v
                          + b1v[:, :half], 0.0)
        zhi = jnp.maximum((ahi[0] + ahi[1] + hhi_ref[...]) * dis_v
                          + b1v[:, half:], 0.0)
        w2 = w2_ref[...]
        h2 = (jnp.dot(zlo, w2[:half], preferred_element_type=jnp.float32)
              + jnp.dot(zhi, w2[half:], preferred_element_type=jnp.float32))
        out_ref[...] = h2 * dis_v

    acc_spec = pl.BlockSpec((_NC, r_blk, half), lambda i: (0, i, 0))
    row_spec = pl.BlockSpec((r_blk, half), lambda i: (i, 0))
    return pl.pallas_call(
        body,
        grid=(grid,),
        in_specs=[
            acc_spec,
            acc_spec,
            row_spec,
            row_spec,
            pl.BlockSpec((r_blk, 1), lambda i: (i, 0)),
            pl.BlockSpec((1, 2 * half), lambda i: (0, 0)),
            pl.BlockSpec((2 * half, d_out), lambda i: (0, 0)),
        ],
        out_specs=pl.BlockSpec((r_blk, d_out), lambda i: (i, 0)),
        out_shape=jax.ShapeDtypeStruct((np_, d_out), jnp.float32),
    )(acc_lo, acc_hi, h_lo, h_hi, dis, b1, W2)


def _tc_final(acc2, h2, dis, b2, r_blk):
    np_, d_out = h2.shape
    grid = np_ // r_blk

    def body(a_ref, h_ref, dis_ref, b_ref, out_ref):
        a = a_ref[...]
        out_ref[...] = (a[0] + a[1] + h_ref[...]) * dis_ref[...] + b_ref[...]

    return pl.pallas_call(
        body,
        grid=(grid,),
        in_specs=[
            pl.BlockSpec((_NC, r_blk, d_out), lambda i: (0, i, 0)),
            pl.BlockSpec((r_blk, d_out), lambda i: (i, 0)),
            pl.BlockSpec((r_blk, 1), lambda i: (i, 0)),
            pl.BlockSpec((1, d_out), lambda i: (0, 0)),
        ],
        out_specs=pl.BlockSpec((r_blk, d_out), lambda i: (i, 0)),
        out_shape=jax.ShapeDtypeStruct((np_, d_out), jnp.float32),
    )(acc2, h2, dis, b2)


def kernel(x, edge_index, W1, b1, W2, b2):
    n, d_in = x.shape
    d_h = W1.shape[1]
    e = edge_index.shape[1]
    assert d_h % 256 == 0 and d_in % 128 == 0
    np_ = -(-n // 2048) * 2048
    r_blk = 1024
    nw = _NC * _NS

    niter = -(-e // (nw * _IB * _CHUNK)) * _IB
    e_pad = nw * niter * _CHUNK
    src = edge_index[0]
    dst = edge_index[1]
    src2 = jnp.concatenate(
        [src, jnp.zeros((e_pad - e,), jnp.int32)]).reshape(-1, _CHUNK)
    pad_dst = n + jnp.arange(e_pad - e, dtype=jnp.int32) % (np_ - n)
    dst2 = jnp.concatenate([dst, pad_dst]).reshape(-1, _CHUNK)
    zeros_flat = jnp.zeros((np_,), jnp.float32)
    zeros_slab = jnp.zeros((np_ // _NS, d_h // 2), jnp.float32)

    deg = _sc_degree(dst2, zeros_flat, np_)
    deg3 = deg.reshape(_NC, np_, 1)
    x_pad = jnp.pad(x, ((0, np_ - n), (0, 0)))
    h_lo, h_hi, dis = _tc_layer1(x_pad, W1, deg3, r_blk)
    acc_lo = _sc_scatter(h_lo, src2, dst2, zeros_slab, np_)
    acc_hi = _sc_scatter(h_hi, src2, dst2, zeros_slab, np_)
    h2 = _tc_layer2(acc_lo, acc_hi, h_lo, h_hi, dis,
                    b1.reshape(1, -1), W2, r_blk)
    acc2 = _sc_scatter(h2, src2, dst2, zeros_slab, np_)
    out = _tc_final(acc2, h2, dis, b2.reshape(1, -1), r_blk)
    return out[:n]


# speedup vs baseline: 1.2219x; 1.2219x over previous
import functools

import jax
import jax.numpy as jnp
from jax import lax
from jax.experimental import pallas as pl
from jax.experimental.pallas import tpu as pltpu
from jax.experimental.pallas import tpu_sc as plsc

_NC = 2
_NS = 16
_CHUNK = 128
_IB = 16
_LANES = 16


def _sc_mesh():
    return plsc.VectorSubcoreMesh(core_axis_name="c", subcore_axis_name="s")


def _sc_degree(dst2, zeros_flat, np_):
    nchunk = dst2.shape[0]
    nw = _NC * _NS
    niter = nchunk // nw
    npw = np_ // _NS

    @functools.partial(
        pl.kernel,
        out_type=jax.ShapeDtypeStruct((_NC, np_), jnp.float32),
        mesh=_sc_mesh(),
        compiler_params=pltpu.CompilerParams(needs_layout_passes=False),
        scratch_types=[
            pltpu.VMEM((niter, _CHUNK), jnp.int32),
            pltpu.VMEM((np_,), jnp.float32),
            pltpu.VMEM((_NS, npw), jnp.float32),
            pltpu.VMEM_SHARED((_NS, np_), jnp.float32),
        ],
    )
    def k(dst_hbm, z_hbm, out_hbm, idx_all, hist_v, red_v, stage_sh):
        c = lax.axis_index("c")
        s = lax.axis_index("s")
        w = c * _NS + s
        pltpu.sync_copy(z_hbm, hist_v)
        pltpu.sync_copy(dst_hbm.at[pl.ds(w * niter, niter)], idx_all)
        ones = jnp.full((_LANES,), 1.0, jnp.float32)

        def chunk_body(i, carry):
            for j in range(_CHUNK // _LANES):
                dvec = idx_all[i, pl.ds(j * _LANES, _LANES)]
                plsc.addupdate_scatter(hist_v, (dvec,), ones)
            return carry

        lax.fori_loop(0, niter, chunk_body, 0)

        pltpu.sync_copy(hist_v, stage_sh.at[s])
        plsc.subcore_barrier()
        for r in range(_NS):
            pltpu.sync_copy(stage_sh.at[r, pl.ds(s * npw, npw)], red_v.at[r])

        def red_body(j, carry):
            acc = red_v[0, pl.ds(j * _LANES, _LANES)]
            for r in range(1, _NS):
                acc = acc + red_v[r, pl.ds(j * _LANES, _LANES)]
            hist_v[pl.ds(j * _LANES, _LANES)] = acc
            return carry

        lax.fori_loop(0, npw // _LANES, red_body, 0)
        pltpu.sync_copy(hist_v.at[pl.ds(0, npw)],
                        out_hbm.at[c, pl.ds(s * npw, npw)])

    return k(dst2, zeros_flat)


def _sc_scatter(h, src1, dst1, zeros_slab, np_):
    nchunk = src1.shape[0] // _CHUNK
    dh = h.shape[1]
    nw = _NC * _NS
    niter = nchunk // nw
    rows_per = np_ // _NS
    ib = _IB

    @functools.partial(
        pl.kernel,
        out_type=jax.ShapeDtypeStruct((_NC, np_, dh), jnp.float32),
        mesh=_sc_mesh(),
        compiler_params=pltpu.CompilerParams(needs_layout_passes=False),
        scratch_types=[
            pltpu.VMEM((_CHUNK,), jnp.int32),
            pltpu.VMEM((_CHUNK,), jnp.int32),
            pltpu.VMEM((_CHUNK,), jnp.int32),
            pltpu.VMEM((_CHUNK,), jnp.int32),
            pltpu.VMEM((_CHUNK, dh), jnp.float32),
            pltpu.VMEM((_CHUNK, dh), jnp.float32),
            pltpu.VMEM_SHARED((np_, dh), jnp.float32),
            pltpu.SemaphoreType.DMA,
            pltpu.SemaphoreType.DMA,
        ],
    )
    def k(h_hbm, src_hbm, dst_hbm, z_hbm, out_hbm, isrc0, idst0, isrc1,
          idst1, rows0, rows1, acc_sh, sem0, sem1):
        c = lax.axis_index("c")
        s = lax.axis_index("s")
        w = c * _NS + s
        base = w * niter * _CHUNK
        half = niter // 2
        pltpu.sync_copy(z_hbm, acc_sh.at[pl.ds(s * rows_per, rows_per)])
        plsc.subcore_barrier()

        pltpu.sync_copy(src_hbm.at[pl.ds(base, _CHUNK)], isrc0)
        pltpu.sync_copy(dst_hbm.at[pl.ds(base, _CHUNK)], idst0)
        pltpu.async_copy(h_hbm.at[isrc0], rows0, sem0)

        def body(i, carry):
            off1 = base + (2 * i + 1) * _CHUNK
            pltpu.sync_copy(src_hbm.at[pl.ds(off1, _CHUNK)], isrc1)
            pltpu.sync_copy(dst_hbm.at[pl.ds(off1, _CHUNK)], idst1)
            pltpu.async_copy(h_hbm.at[isrc1], rows1, sem1)
            pltpu.make_async_copy(h_hbm.at[pl.ds(0, _CHUNK)], rows0,
                                  sem0).wait()
            pltpu.sync_copy(rows0, acc_sh.at[idst0], add=True)

            @pl.when(i < half - 1)
            def _():
                off2 = base + (2 * i + 2) * _CHUNK
                pltpu.sync_copy(src_hbm.at[pl.ds(off2, _CHUNK)], isrc0)
                pltpu.sync_copy(dst_hbm.at[pl.ds(off2, _CHUNK)], idst0)
                pltpu.async_copy(h_hbm.at[isrc0], rows0, sem0)

            pltpu.make_async_copy(h_hbm.at[pl.ds(0, _CHUNK)], rows1,
                                  sem1).wait()
            pltpu.sync_copy(rows1, acc_sh.at[idst1], add=True)
            return carry

        lax.fori_loop(0, half, body, 0)
        plsc.subcore_barrier()
        pltpu.sync_copy(acc_sh.at[pl.ds(s * rows_per, rows_per)],
                        out_hbm.at[c, pl.ds(s * rows_per, rows_per)])

    return k(h, src1, dst1, zeros_slab)


def _tc_layer1(x_pad, W1, deg3, r_blk):
    np_, d_in = x_pad.shape
    d_h = W1.shape[1]
    half = d_h // 2
    grid = np_ // r_blk

    def body(x_ref, w_ref, deg_ref, lo_ref, hi_ref, dis_ref):
        deg = deg_ref[...]
        dis = lax.rsqrt(deg[0] + deg[1] + 1.0)
        h = jnp.dot(x_ref[...], w_ref[...], preferred_element_type=jnp.float32)
        lo_ref[...] = h[:, :half] * dis
        hi_ref[...] = h[:, half:] * dis
        dis_ref[...] = dis

    return pl.pallas_call(
        body,
        grid=(grid,),
        in_specs=[
            pl.BlockSpec((r_blk, d_in), lambda i: (i, 0)),
            pl.BlockSpec((d_in, d_h), lambda i: (0, 0)),
            pl.BlockSpec((_NC, r_blk, 1), lambda i: (0, i, 0)),
        ],
        out_specs=[
            pl.BlockSpec((r_blk, half), lambda i: (i, 0)),
            pl.BlockSpec((r_blk, half), lambda i: (i, 0)),
            pl.BlockSpec((r_blk, 1), lambda i: (i, 0)),
        ],
        out_shape=[
            jax.ShapeDtypeStruct((np_, half), jnp.float32),
            jax.ShapeDtypeStruct((np_, half), jnp.float32),
            jax.ShapeDtypeStruct((np_, 1), jnp.float32),
        ],
    )(x_pad, W1, deg3)


def _tc_layer2(acc_lo, acc_hi, h_lo, h_hi, dis, b1, W2, r_blk):
    np_ = h_lo.shape[0]
    half = h_lo.shape[1]
    d_out = W2.shape[1]
    grid = np_ // r_blk

    def body(alo_ref, ahi_ref, hlo_ref, hhi_ref, dis_ref, b1_ref, w2_ref,
             out_ref):
        dis_v = dis_ref[...]
        alo = alo_ref[...]
        ahi = ahi_ref[...]
        b1v = b1_ref[...]
        zlo = jnp.maximum((alo[0] + alo[1] + hlo_ref[...]) * dis_v
                          + b1v[:, :half], 0.0)
        zhi = jnp.maximum((ahi[0] + ahi[1] + hhi_ref[...]) * dis_v
                          + b1v[:, half:], 0.0)
        w2 = w2_ref[...]
        h2 = (jnp.dot(zlo, w2[:half], preferred_element_type=jnp.float32)
              + jnp.dot(zhi, w2[half:], preferred_element_type=jnp.float32))
        out_ref[...] = h2 * dis_v

    acc_spec = pl.BlockSpec((_NC, r_blk, half), lambda i: (0, i, 0))
    row_spec = pl.BlockSpec((r_blk, half), lambda i: (i, 0))
    return pl.pallas_call(
        body,
        grid=(grid,),
        in_specs=[
            acc_spec,
            acc_spec,
            row_spec,
            row_spec,
            pl.BlockSpec((r_blk, 1), lambda i: (i, 0)),
            pl.BlockSpec((1, 2 * half), lambda i: (0, 0)),
            pl.BlockSpec((2 * half, d_out), lambda i: (0, 0)),
        ],
        out_specs=pl.BlockSpec((r_blk, d_out), lambda i: (i, 0)),
        out_shape=jax.ShapeDtypeStruct((np_, d_out), jnp.float32),
    )(acc_lo, acc_hi, h_lo, h_hi, dis, b1, W2)


def _tc_final(acc2, h2, dis, b2, r_blk):
    np_, d_out = h2.shape
    grid = np_ // r_blk

    def body(a_ref, h_ref, dis_ref, b_ref, out_ref):
        a = a_ref[...]
        out_ref[...] = (a[0] + a[1] + h_ref[...]) * dis_ref[...] + b_ref[...]

    return pl.pallas_call(
        body,
        grid=(grid,),
        in_specs=[
            pl.BlockSpec((_NC, r_blk, d_out), lambda i: (0, i, 0)),
            pl.BlockSpec((r_blk, d_out), lambda i: (i, 0)),
            pl.BlockSpec((r_blk, 1), lambda i: (i, 0)),
            pl.BlockSpec((1, d_out), lambda i: (0, 0)),
        ],
        out_specs=pl.BlockSpec((r_blk, d_out), lambda i: (i, 0)),
        out_shape=jax.ShapeDtypeStruct((np_, d_out), jnp.float32),
    )(acc2, h2, dis, b2)


def kernel(x, edge_index, W1, b1, W2, b2):
    n, d_in = x.shape
    d_h = W1.shape[1]
    e = edge_index.shape[1]
    assert d_h % 256 == 0 and d_in % 128 == 0
    np_ = -(-n // 2048) * 2048
    r_blk = 1024
    nw = _NC * _NS

    niter = -(-e // (nw * _IB * _CHUNK)) * _IB
    e_pad = nw * niter * _CHUNK
    src = edge_index[0]
    dst = edge_index[1]
    src1 = jnp.concatenate([src, jnp.zeros((e_pad - e,), jnp.int32)])
    pad_dst = n + jnp.arange(e_pad - e, dtype=jnp.int32) % (np_ - n)
    dst1 = jnp.concatenate([dst, pad_dst])
    dst2 = dst1.reshape(-1, _CHUNK)
    zeros_flat = jnp.zeros((np_,), jnp.float32)
    zeros_slab = jnp.zeros((np_ // _NS, d_h // 2), jnp.float32)

    deg = _sc_degree(dst2, zeros_flat, np_)
    deg3 = deg.reshape(_NC, np_, 1)
    x_pad = jnp.pad(x, ((0, np_ - n), (0, 0)))
    h_lo, h_hi, dis = _tc_layer1(x_pad, W1, deg3, r_blk)
    acc_lo = _sc_scatter(h_lo, src1, dst1, zeros_slab, np_)
    acc_hi = _sc_scatter(h_hi, src1, dst1, zeros_slab, np_)
    h2 = _tc_layer2(acc_lo, acc_hi, h_lo, h_hi, dis,
                    b1.reshape(1, -1), W2, r_blk)
    acc2 = _sc_scatter(h2, src1, dst1, zeros_slab, np_)
    out = _tc_final(acc2, h2, dis, b2.reshape(1, -1), r_blk)
    return out[:n]


# speedup vs baseline: 3.2073x; 2.6249x over previous
import functools

import jax
import jax.numpy as jnp
from jax import lax
from jax.experimental import pallas as pl
from jax.experimental.pallas import tpu as pltpu
from jax.experimental.pallas import tpu_sc as plsc

_NC = 2
_NS = 16
_CHUNK = 128
_IB = 16
_LANES = 16


def _sc_mesh():
    return plsc.VectorSubcoreMesh(core_axis_name="c", subcore_axis_name="s")


def _sc_degree(dst2, zeros_flat, np_):
    nchunk = dst2.shape[0]
    nw = _NC * _NS
    niter = nchunk // nw
    npw = np_ // _NS

    @functools.partial(
        pl.kernel,
        out_type=jax.ShapeDtypeStruct((_NC, np_), jnp.float32),
        mesh=_sc_mesh(),
        compiler_params=pltpu.CompilerParams(needs_layout_passes=False),
        scratch_types=[
            pltpu.VMEM((niter, _CHUNK), jnp.int32),
            pltpu.VMEM((np_,), jnp.float32),
            pltpu.VMEM((_NS, npw), jnp.float32),
            pltpu.VMEM_SHARED((_NS, np_), jnp.float32),
        ],
    )
    def k(dst_hbm, z_hbm, out_hbm, idx_all, hist_v, red_v, stage_sh):
        c = lax.axis_index("c")
        s = lax.axis_index("s")
        w = c * _NS + s
        pltpu.sync_copy(z_hbm, hist_v)
        pltpu.sync_copy(dst_hbm.at[pl.ds(w * niter, niter)], idx_all)
        ones = jnp.full((_LANES,), 1.0, jnp.float32)

        def chunk_body(i, carry):
            for j in range(_CHUNK // _LANES):
                dvec = idx_all[i, pl.ds(j * _LANES, _LANES)]
                plsc.addupdate_scatter(hist_v, (dvec,), ones)
            return carry

        lax.fori_loop(0, niter, chunk_body, 0)

        pltpu.sync_copy(hist_v, stage_sh.at[s])
        plsc.subcore_barrier()
        for r in range(_NS):
            pltpu.sync_copy(stage_sh.at[r, pl.ds(s * npw, npw)], red_v.at[r])

        def red_body(j, carry):
            acc = red_v[0, pl.ds(j * _LANES, _LANES)]
            for r in range(1, _NS):
                acc = acc + red_v[r, pl.ds(j * _LANES, _LANES)]
            hist_v[pl.ds(j * _LANES, _LANES)] = acc
            return carry

        lax.fori_loop(0, npw // _LANES, red_body, 0)
        pltpu.sync_copy(hist_v.at[pl.ds(0, npw)],
                        out_hbm.at[c, pl.ds(s * npw, npw)])

    return k(dst2, zeros_flat)


def _sc_scatter(h, src1, dst1, zeros_slab, np_):
    nchunk = src1.shape[0] // _CHUNK
    dh = h.shape[1]
    nw = _NC * _NS
    niter = nchunk // nw
    rows_per = np_ // _NS
    ib = _IB

    @functools.partial(
        pl.kernel,
        out_type=jax.ShapeDtypeStruct((_NC, np_, dh), jnp.float32),
        mesh=_sc_mesh(),
        compiler_params=pltpu.CompilerParams(needs_layout_passes=False),
        scratch_types=[
            pltpu.VMEM((_CHUNK,), jnp.int32),
            pltpu.VMEM((_CHUNK,), jnp.int32),
            pltpu.VMEM((_CHUNK,), jnp.int32),
            pltpu.VMEM((_CHUNK,), jnp.int32),
            pltpu.VMEM((_CHUNK, dh), jnp.float32),
            pltpu.VMEM((_CHUNK, dh), jnp.float32),
            pltpu.VMEM_SHARED((np_, dh), jnp.float32),
            pltpu.SemaphoreType.DMA,
            pltpu.SemaphoreType.DMA,
        ],
    )
    def k(h_hbm, src_hbm, dst_hbm, z_hbm, out_hbm, isrc0, idst0, isrc1,
          idst1, rows0, rows1, acc_sh, sem0, sem1):
        c = lax.axis_index("c")
        s = lax.axis_index("s")
        w = c * _NS + s
        base = w * niter * _CHUNK
        half = niter // 2
        pltpu.sync_copy(z_hbm, acc_sh.at[pl.ds(s * rows_per, rows_per)])
        plsc.subcore_barrier()

        pltpu.sync_copy(src_hbm.at[pl.ds(base, _CHUNK)], isrc0)
        pltpu.sync_copy(dst_hbm.at[pl.ds(base, _CHUNK)], idst0)
        pltpu.async_copy(h_hbm.at[isrc0], rows0, sem0)

        def body(i, carry):
            off1 = base + (2 * i + 1) * _CHUNK
            pltpu.sync_copy(src_hbm.at[pl.ds(off1, _CHUNK)], isrc1)
            pltpu.sync_copy(dst_hbm.at[pl.ds(off1, _CHUNK)], idst1)
            pltpu.async_copy(h_hbm.at[isrc1], rows1, sem1)
            pltpu.make_async_copy(h_hbm.at[pl.ds(0, _CHUNK)], rows0,
                                  sem0).wait()
            pltpu.sync_copy(rows0, acc_sh.at[idst0], add=True)

            @pl.when(i < half - 1)
            def _():
                off2 = base + (2 * i + 2) * _CHUNK
                pltpu.sync_copy(src_hbm.at[pl.ds(off2, _CHUNK)], isrc0)
                pltpu.sync_copy(dst_hbm.at[pl.ds(off2, _CHUNK)], idst0)
                pltpu.async_copy(h_hbm.at[isrc0], rows0, sem0)

            pltpu.make_async_copy(h_hbm.at[pl.ds(0, _CHUNK)], rows1,
                                  sem1).wait()
            pltpu.sync_copy(rows1, acc_sh.at[idst1], add=True)
            return carry

        lax.fori_loop(0, half, body, 0)
        plsc.subcore_barrier()
        pltpu.sync_copy(acc_sh.at[pl.ds(s * rows_per, rows_per)],
                        out_hbm.at[c, pl.ds(s * rows_per, rows_per)])

    return k(h, src1, dst1, zeros_slab)


def _tc_layer1(x_pad, W1, deg3, r_blk):
    np_, d_in = x_pad.shape
    d_h = W1.shape[1]
    half = d_h // 2
    grid = np_ // r_blk

    def body(x_ref, w_ref, deg_ref, lo_ref, hi_ref, dis_ref):
        deg = deg_ref[...]
        dis = lax.rsqrt(deg[0] + deg[1] + 1.0)
        h = jnp.dot(x_ref[...], w_ref[...], preferred_element_type=jnp.float32)
        lo_ref[...] = h[:, :half] * dis
        hi_ref[...] = h[:, half:] * dis
        dis_ref[...] = dis

    return pl.pallas_call(
        body,
        grid=(grid,),
        in_specs=[
            pl.BlockSpec((r_blk, d_in), lambda i: (i, 0)),
            pl.BlockSpec((d_in, d_h), lambda i: (0, 0)),
            pl.BlockSpec((_NC, r_blk, 1), lambda i: (0, i, 0)),
        ],
        out_specs=[
            pl.BlockSpec((r_blk, half), lambda i: (i, 0)),
            pl.BlockSpec((r_blk, half), lambda i: (i, 0)),
            pl.BlockSpec((r_blk, 1), lambda i: (i, 0)),
        ],
        out_shape=[
            jax.ShapeDtypeStruct((np_, half), jnp.float32),
            jax.ShapeDtypeStruct((np_, half), jnp.float32),
            jax.ShapeDtypeStruct((np_, 1), jnp.float32),
        ],
    )(x_pad, W1, deg3)


def _tc_layer2(acc_lo, acc_hi, h_lo, h_hi, dis, b1, W2, r_blk):
    np_ = h_lo.shape[0]
    half = h_lo.shape[1]
    d_out = W2.shape[1]
    grid = np_ // r_blk

    def body(alo_ref, ahi_ref, hlo_ref, hhi_ref, dis_ref, b1_ref, w2_ref,
             out_ref):
        dis_v = dis_ref[...]
        alo = alo_ref[...]
        ahi = ahi_ref[...]
        b1v = b1_ref[...]
        zlo = jnp.maximum((alo[0] + alo[1] + hlo_ref[...]) * dis_v
                          + b1v[:, :half], 0.0)
        zhi = jnp.maximum((ahi[0] + ahi[1] + hhi_ref[...]) * dis_v
                          + b1v[:, half:], 0.0)
        w2 = w2_ref[...]
        h2 = (jnp.dot(zlo, w2[:half], preferred_element_type=jnp.float32)
              + jnp.dot(zhi, w2[half:], preferred_element_type=jnp.float32))
        out_ref[...] = h2 * dis_v

    acc_spec = pl.BlockSpec((_NC, r_blk, half), lambda i: (0, i, 0))
    row_spec = pl.BlockSpec((r_blk, half), lambda i: (i, 0))
    return pl.pallas_call(
        body,
        grid=(grid,),
        in_specs=[
            acc_spec,
            acc_spec,
            row_spec,
            row_spec,
            pl.BlockSpec((r_blk, 1), lambda i: (i, 0)),
            pl.BlockSpec((1, 2 * half), lambda i: (0, 0)),
            pl.BlockSpec((2 * half, d_out), lambda i: (0, 0)),
        ],
        out_specs=pl.BlockSpec((r_blk, d_out), lambda i: (i, 0)),
        out_shape=jax.ShapeDtypeStruct((np_, d_out), jnp.float32),
    )(acc_lo, acc_hi, h_lo, h_hi, dis, b1, W2)


def _tc_final(acc2, h2, dis, b2, r_blk):
    np_, d_out = h2.shape
    grid = np_ // r_blk

    def body(a_ref, h_ref, dis_ref, b_ref, out_ref):
        a = a_ref[...]
        out_ref[...] = (a[0] + a[1] + h_ref[...]) * dis_ref[...] + b_ref[...]

    return pl.pallas_call(
        body,
        grid=(grid,),
        in_specs=[
            pl.BlockSpec((_NC, r_blk, d_out), lambda i: (0, i, 0)),
            pl.BlockSpec((r_blk, d_out), lambda i: (i, 0)),
            pl.BlockSpec((r_blk, 1), lambda i: (i, 0)),
            pl.BlockSpec((1, d_out), lambda i: (0, 0)),
        ],
        out_specs=pl.BlockSpec((r_blk, d_out), lambda i: (i, 0)),
        out_shape=jax.ShapeDtypeStruct((np_, d_out), jnp.float32),
    )(acc2, h2, dis, b2)


def kernel(x, edge_index, W1, b1, W2, b2):
    n, d_in = x.shape
    d_h = W1.shape[1]
    e = edge_index.shape[1]
    assert d_h % 256 == 0 and d_in % 128 == 0
    np_ = -(-n // 2048) * 2048
    r_blk = 1024
    nw = _NC * _NS

    niter = -(-e // (nw * _IB * _CHUNK)) * _IB
    e_pad = nw * niter * _CHUNK
    src = edge_index[0]
    dst = edge_index[1]
    pad_src = jnp.arange(e_pad - e, dtype=jnp.int32) % n
    src1 = jnp.concatenate([src, pad_src])
    pad_dst = n + jnp.arange(e_pad - e, dtype=jnp.int32) % (np_ - n)
    dst1 = jnp.concatenate([dst, pad_dst])
    dst2 = dst1.reshape(-1, _CHUNK)
    zeros_flat = jnp.zeros((np_,), jnp.float32)
    zeros_slab = jnp.zeros((np_ // _NS, d_h // 2), jnp.float32)

    deg = _sc_degree(dst2, zeros_flat, np_)
    deg3 = deg.reshape(_NC, np_, 1)
    x_pad = jnp.pad(x, ((0, np_ - n), (0, 0)))
    h_lo, h_hi, dis = _tc_layer1(x_pad, W1, deg3, r_blk)
    acc_lo = _sc_scatter(h_lo, src1, dst1, zeros_slab, np_)
    acc_hi = _sc_scatter(h_hi, src1, dst1, zeros_slab, np_)
    h2 = _tc_layer2(acc_lo, acc_hi, h_lo, h_hi, dis,
                    b1.reshape(1, -1), W2, r_blk)
    acc2 = _sc_scatter(h2, src1, dst1, zeros_slab, np_)
    out = _tc_final(acc2, h2, dis, b2.reshape(1, -1), r_blk)
    return out[:n]


# speedup vs baseline: 3.3422x; 1.0421x over previous
import functools

import jax
import jax.numpy as jnp
from jax import lax
from jax.experimental import pallas as pl
from jax.experimental.pallas import tpu as pltpu
from jax.experimental.pallas import tpu_sc as plsc

_NC = 2
_NS = 16
_CHUNK = 128
_IB = 16
_LANES = 16


def _sc_mesh():
    return plsc.VectorSubcoreMesh(core_axis_name="c", subcore_axis_name="s")


def _sc_degree(dst2, zeros_flat, np_):
    nchunk = dst2.shape[0]
    nw = _NC * _NS
    niter = nchunk // nw
    npw = np_ // _NS

    @functools.partial(
        pl.kernel,
        out_type=jax.ShapeDtypeStruct((_NC, np_), jnp.float32),
        mesh=_sc_mesh(),
        compiler_params=pltpu.CompilerParams(needs_layout_passes=False),
        scratch_types=[
            pltpu.VMEM((niter, _CHUNK), jnp.int32),
            pltpu.VMEM((np_,), jnp.float32),
            pltpu.VMEM((_NS, npw), jnp.float32),
            pltpu.VMEM_SHARED((_NS, np_), jnp.float32),
        ],
    )
    def k(dst_hbm, z_hbm, out_hbm, idx_all, hist_v, red_v, stage_sh):
        c = lax.axis_index("c")
        s = lax.axis_index("s")
        w = c * _NS + s
        pltpu.sync_copy(z_hbm, hist_v)
        pltpu.sync_copy(dst_hbm.at[pl.ds(w * niter, niter)], idx_all)
        ones = jnp.full((_LANES,), 1.0, jnp.float32)

        def chunk_body(i, carry):
            for j in range(_CHUNK // _LANES):
                dvec = idx_all[i, pl.ds(j * _LANES, _LANES)]
                plsc.addupdate_scatter(hist_v, (dvec,), ones)
            return carry

        lax.fori_loop(0, niter, chunk_body, 0)

        pltpu.sync_copy(hist_v, stage_sh.at[s])
        plsc.subcore_barrier()
        for r in range(_NS):
            pltpu.sync_copy(stage_sh.at[r, pl.ds(s * npw, npw)], red_v.at[r])

        def red_body(j, carry):
            acc = red_v[0, pl.ds(j * _LANES, _LANES)]
            for r in range(1, _NS):
                acc = acc + red_v[r, pl.ds(j * _LANES, _LANES)]
            hist_v[pl.ds(j * _LANES, _LANES)] = acc
            return carry

        lax.fori_loop(0, npw // _LANES, red_body, 0)
        pltpu.sync_copy(hist_v.at[pl.ds(0, npw)],
                        out_hbm.at[c, pl.ds(s * npw, npw)])

    return k(dst2, zeros_flat)


def _sc_scatter(h, src2, dst2, zeros_slab, np_):
    nchunk = src2.shape[0]
    dh = h.shape[1]
    nw = _NC * _NS
    niter = nchunk // nw
    rows_per = np_ // _NS
    ib = _IB

    @functools.partial(
        pl.kernel,
        out_type=jax.ShapeDtypeStruct((_NC, np_, dh), jnp.float32),
        mesh=_sc_mesh(),
        compiler_params=pltpu.CompilerParams(needs_layout_passes=False),
        scratch_types=[
            pltpu.VMEM((_IB, _CHUNK), jnp.int32),
            pltpu.VMEM((_IB, _CHUNK), jnp.int32),
            pltpu.VMEM((_CHUNK, dh), jnp.float32),
            pltpu.VMEM((_CHUNK, dh), jnp.float32),
            pltpu.VMEM_SHARED((np_, dh), jnp.float32),
            pltpu.SemaphoreType.DMA,
            pltpu.SemaphoreType.DMA,
            pltpu.SemaphoreType.DMA,
            pltpu.SemaphoreType.DMA,
        ],
    )
    def k(h_hbm, src_hbm, dst_hbm, z_hbm, out_hbm, isrc, idst, rows0, rows1,
          acc_sh, semg0, semg1, sems0, sems1):
        c = lax.axis_index("c")
        s = lax.axis_index("s")
        w = c * _NS + s
        pltpu.sync_copy(z_hbm, acc_sh.at[pl.ds(s * rows_per, rows_per)])
        plsc.subcore_barrier()

        def wait_g0():
            pltpu.make_async_copy(h_hbm.at[pl.ds(0, _CHUNK)], rows0,
                                  semg0).wait()

        def wait_g1():
            pltpu.make_async_copy(h_hbm.at[pl.ds(0, _CHUNK)], rows1,
                                  semg1).wait()

        def wait_s0():
            pltpu.make_async_copy(h_hbm.at[pl.ds(0, _CHUNK)], rows0,
                                  sems0).wait()

        def wait_s1():
            pltpu.make_async_copy(h_hbm.at[pl.ds(0, _CHUNK)], rows1,
                                  sems1).wait()

        def outer(o, carry):
            base = w * niter + o * _IB
            pltpu.sync_copy(src_hbm.at[pl.ds(base, _IB)], isrc)
            pltpu.sync_copy(dst_hbm.at[pl.ds(base, _IB)], idst)
            pltpu.async_copy(h_hbm.at[isrc.at[0]], rows0, semg0)

            def body(i, carry2):
                wait_g0()
                pltpu.async_copy(rows0, acc_sh.at[idst.at[2 * i]], sems0,
                                 add=True)

                @pl.when(i > 0)
                def _():
                    wait_s1()

                pltpu.async_copy(h_hbm.at[isrc.at[2 * i + 1]], rows1, semg1)
                wait_s0()

                @pl.when(i < _IB // 2 - 1)
                def _():
                    pltpu.async_copy(h_hbm.at[isrc.at[2 * i + 2]], rows0,
                                     semg0)

                wait_g1()
                pltpu.async_copy(rows1, acc_sh.at[idst.at[2 * i + 1]], sems1,
                                 add=True)
                return carry2

            lax.fori_loop(0, _IB // 2, body, 0)
            wait_s1()
            return carry

        lax.fori_loop(0, niter // _IB, outer, 0)
        plsc.subcore_barrier()
        pltpu.sync_copy(acc_sh.at[pl.ds(s * rows_per, rows_per)],
                        out_hbm.at[c, pl.ds(s * rows_per, rows_per)])

    return k(h, src2, dst2, zeros_slab)


def _tc_layer1(x_pad, W1, deg3, r_blk):
    np_, d_in = x_pad.shape
    d_h = W1.shape[1]
    half = d_h // 2
    grid = np_ // r_blk

    def body(x_ref, w_ref, deg_ref, lo_ref, hi_ref, dis_ref):
        deg = deg_ref[...]
        dis = lax.rsqrt(deg[0] + deg[1] + 1.0)
        h = jnp.dot(x_ref[...], w_ref[...], preferred_element_type=jnp.float32)
        lo_ref[...] = h[:, :half] * dis
        hi_ref[...] = h[:, half:] * dis
        dis_ref[...] = dis

    return pl.pallas_call(
        body,
        grid=(grid,),
        in_specs=[
            pl.BlockSpec((r_blk, d_in), lambda i: (i, 0)),
            pl.BlockSpec((d_in, d_h), lambda i: (0, 0)),
            pl.BlockSpec((_NC, r_blk, 1), lambda i: (0, i, 0)),
        ],
        out_specs=[
            pl.BlockSpec((r_blk, half), lambda i: (i, 0)),
            pl.BlockSpec((r_blk, half), lambda i: (i, 0)),
            pl.BlockSpec((r_blk, 1), lambda i: (i, 0)),
        ],
        out_shape=[
            jax.ShapeDtypeStruct((np_, half), jnp.float32),
            jax.ShapeDtypeStruct((np_, half), jnp.float32),
            jax.ShapeDtypeStruct((np_, 1), jnp.float32),
        ],
    )(x_pad, W1, deg3)


def _tc_layer2(acc_lo, acc_hi, h_lo, h_hi, dis, b1, W2, r_blk):
    np_ = h_lo.shape[0]
    half = h_lo.shape[1]
    d_out = W2.shape[1]
    grid = np_ // r_blk

    def body(alo_ref, ahi_ref, hlo_ref, hhi_ref, dis_ref, b1_ref, w2_ref,
             out_ref):
        dis_v = dis_ref[...]
        alo = alo_ref[...]
        ahi = ahi_ref[...]
        b1v = b1_ref[...]
        zlo = jnp.maximum((alo[0] + alo[1] + hlo_ref[...]) * dis_v
                          + b1v[:, :half], 0.0)
        zhi = jnp.maximum((ahi[0] + ahi[1] + hhi_ref[...]) * dis_v
                          + b1v[:, half:], 0.0)
        w2 = w2_ref[...]
        h2 = (jnp.dot(zlo, w2[:half], preferred_element_type=jnp.float32)
              + jnp.dot(zhi, w2[half:], preferred_element_type=jnp.float32))
        out_ref[...] = h2 * dis_v

    acc_spec = pl.BlockSpec((_NC, r_blk, half), lambda i: (0, i, 0))
    row_spec = pl.BlockSpec((r_blk, half), lambda i: (i, 0))
    return pl.pallas_call(
        body,
        grid=(grid,),
        in_specs=[
            acc_spec,
            acc_spec,
            row_spec,
            row_spec,
            pl.BlockSpec((r_blk, 1), lambda i: (i, 0)),
            pl.BlockSpec((1, 2 * half), lambda i: (0, 0)),
            pl.BlockSpec((2 * half, d_out), lambda i: (0, 0)),
        ],
        out_specs=pl.BlockSpec((r_blk, d_out), lambda i: (i, 0)),
        out_shape=jax.ShapeDtypeStruct((np_, d_out), jnp.float32),
    )(acc_lo, acc_hi, h_lo, h_hi, dis, b1, W2)


def _tc_final(acc2, h2, dis, b2, r_blk):
    np_, d_out = h2.shape
    grid = np_ // r_blk

    def body(a_ref, h_ref, dis_ref, b_ref, out_ref):
        a = a_ref[...]
        out_ref[...] = (a[0] + a[1] + h_ref[...]) * dis_ref[...] + b_ref[...]

    return pl.pallas_call(
        body,
        grid=(grid,),
        in_specs=[
            pl.BlockSpec((_NC, r_blk, d_out), lambda i: (0, i, 0)),
            pl.BlockSpec((r_blk, d_out), lambda i: (i, 0)),
            pl.BlockSpec((r_blk, 1), lambda i: (i, 0)),
            pl.BlockSpec((1, d_out), lambda i: (0, 0)),
        ],
        out_specs=pl.BlockSpec((r_blk, d_out), lambda i: (i, 0)),
        out_shape=jax.ShapeDtypeStruct((np_, d_out), jnp.float32),
    )(acc2, h2, dis, b2)


def kernel(x, edge_index, W1, b1, W2, b2):
    n, d_in = x.shape
    d_h = W1.shape[1]
    e = edge_index.shape[1]
    assert d_h % 256 == 0 and d_in % 128 == 0
    np_ = -(-n // 2048) * 2048
    r_blk = 1024
    nw = _NC * _NS

    niter = -(-e // (nw * _IB * _CHUNK)) * _IB
    e_pad = nw * niter * _CHUNK
    src = edge_index[0]
    dst = edge_index[1]
    pad_src = jnp.arange(e_pad - e, dtype=jnp.int32) % n
    src2 = jnp.concatenate([src, pad_src]).reshape(-1, _CHUNK)
    pad_dst = n + jnp.arange(e_pad - e, dtype=jnp.int32) % (np_ - n)
    dst2 = jnp.concatenate([dst, pad_dst]).reshape(-1, _CHUNK)
    zeros_flat = jnp.zeros((np_,), jnp.float32)
    zeros_slab = jnp.zeros((np_ // _NS, d_h // 2), jnp.float32)

    deg = _sc_degree(dst2, zeros_flat, np_)
    deg3 = deg.reshape(_NC, np_, 1)
    x_pad = jnp.pad(x, ((0, np_ - n), (0, 0)))
    h_lo, h_hi, dis = _tc_layer1(x_pad, W1, deg3, r_blk)
    acc_lo = _sc_scatter(h_lo, src2, dst2, zeros_slab, np_)
    acc_hi = _sc_scatter(h_hi, src2, dst2, zeros_slab, np_)
    h2 = _tc_layer2(acc_lo, acc_hi, h_lo, h_hi, dis,
                    b1.reshape(1, -1), W2, r_blk)
    acc2 = _sc_scatter(h2, src2, dst2, zeros_slab, np_)
    out = _tc_final(acc2, h2, dis, b2.reshape(1, -1), r_blk)
    return out[:n]


# speedup vs baseline: 3.3685x; 1.0079x over previous
import functools

import jax
import jax.numpy as jnp
from jax import lax
from jax.experimental import pallas as pl
from jax.experimental.pallas import tpu as pltpu
from jax.experimental.pallas import tpu_sc as plsc

_NC = 2
_NS = 16
_CHUNK = 128
_IB = 16
_LANES = 16


def _sc_mesh():
    return plsc.VectorSubcoreMesh(core_axis_name="c", subcore_axis_name="s")


def _sc_degree(dst2, zeros_flat, np_):
    nchunk = dst2.shape[0]
    nw = _NC * _NS
    niter = nchunk // nw
    npw = np_ // _NS

    @functools.partial(
        pl.kernel,
        out_type=jax.ShapeDtypeStruct((_NC, np_), jnp.float32),
        mesh=_sc_mesh(),
        compiler_params=pltpu.CompilerParams(needs_layout_passes=False),
        scratch_types=[
            pltpu.VMEM((niter, _CHUNK), jnp.int32),
            pltpu.VMEM((np_,), jnp.float32),
            pltpu.VMEM((_NS, npw), jnp.float32),
            pltpu.VMEM_SHARED((_NS, np_), jnp.float32),
        ],
    )
    def k(dst_hbm, z_hbm, out_hbm, idx_all, hist_v, red_v, stage_sh):
        c = lax.axis_index("c")
        s = lax.axis_index("s")
        w = c * _NS + s
        pltpu.sync_copy(z_hbm, hist_v)
        pltpu.sync_copy(dst_hbm.at[pl.ds(w * niter, niter)], idx_all)
        ones = jnp.full((_LANES,), 1.0, jnp.float32)

        def chunk_body(i, carry):
            for j in range(_CHUNK // _LANES):
                dvec = idx_all[i, pl.ds(j * _LANES, _LANES)]
                plsc.addupdate_scatter(hist_v, (dvec,), ones)
            return carry

        lax.fori_loop(0, niter, chunk_body, 0)

        pltpu.sync_copy(hist_v, stage_sh.at[s])
        plsc.subcore_barrier()
        for r in range(_NS):
            pltpu.sync_copy(stage_sh.at[r, pl.ds(s * npw, npw)], red_v.at[r])

        def red_body(j, carry):
            acc = red_v[0, pl.ds(j * _LANES, _LANES)]
            for r in range(1, _NS):
                acc = acc + red_v[r, pl.ds(j * _LANES, _LANES)]
            hist_v[pl.ds(j * _LANES, _LANES)] = acc
            return carry

        lax.fori_loop(0, npw // _LANES, red_body, 0)
        pltpu.sync_copy(hist_v.at[pl.ds(0, npw)],
                        out_hbm.at[c, pl.ds(s * npw, npw)])

    return k(dst2, zeros_flat)


def _sc_scatter(h_list, src2, dst2, zeros_slab, np_):
    nchunk = src2.shape[0]
    dh = h_list[0].shape[1]
    nw = _NC * _NS
    niter = nchunk // nw
    rows_per = np_ // _NS
    nph = len(h_list)

    k = functools.partial(
        pl.kernel,
        out_type=[jax.ShapeDtypeStruct((_NC, np_, dh), jnp.float32)] * nph,
        mesh=_sc_mesh(),
        compiler_params=pltpu.CompilerParams(needs_layout_passes=False),
        scratch_types=[
            pltpu.VMEM((_IB, _CHUNK), jnp.int32),
            pltpu.VMEM((_IB, _CHUNK), jnp.int32),
            pltpu.VMEM((_CHUNK, dh), jnp.float32),
            pltpu.VMEM((_CHUNK, dh), jnp.float32),
            pltpu.VMEM_SHARED((np_, dh), jnp.float32),
            pltpu.SemaphoreType.DMA,
            pltpu.SemaphoreType.DMA,
            pltpu.SemaphoreType.DMA,
            pltpu.SemaphoreType.DMA,
        ],
    )

    def _impl(h_hbms, src_hbm, dst_hbm, z_hbm, out_hbms, isrc, idst, rows0,
              rows1, acc_sh, semg0, semg1, sems0, sems1):
        c = lax.axis_index("c")
        s = lax.axis_index("s")
        w = c * _NS + s

        def wait(buf, sem):
            pltpu.make_async_copy(h_hbms[0].at[pl.ds(0, _CHUNK)], buf,
                                  sem).wait()

        for h_hbm, out_hbm in zip(h_hbms, out_hbms):
            pltpu.sync_copy(z_hbm, acc_sh.at[pl.ds(s * rows_per, rows_per)])
            plsc.subcore_barrier()

            def outer(o, carry, h_hbm=h_hbm):
                base = w * niter + o * _IB
                pltpu.sync_copy(src_hbm.at[pl.ds(base, _IB)], isrc)
                pltpu.sync_copy(dst_hbm.at[pl.ds(base, _IB)], idst)
                pltpu.async_copy(h_hbm.at[isrc.at[0]], rows0, semg0)

                def body(i, carry2):
                    wait(rows0, semg0)
                    pltpu.async_copy(rows0, acc_sh.at[idst.at[2 * i]], sems0,
                                     add=True)

                    @pl.when(i > 0)
                    def _():
                        wait(rows1, sems1)

                    pltpu.async_copy(h_hbm.at[isrc.at[2 * i + 1]], rows1,
                                     semg1)
                    wait(rows0, sems0)

                    @pl.when(i < _IB // 2 - 1)
                    def _():
                        pltpu.async_copy(h_hbm.at[isrc.at[2 * i + 2]], rows0,
                                         semg0)

                    wait(rows1, semg1)
                    pltpu.async_copy(rows1, acc_sh.at[idst.at[2 * i + 1]],
                                     sems1, add=True)
                    return carry2

                lax.fori_loop(0, _IB // 2, body, 0)
                wait(rows1, sems1)
                return carry

            lax.fori_loop(0, niter // _IB, outer, 0)
            plsc.subcore_barrier()
            pltpu.sync_copy(acc_sh.at[pl.ds(s * rows_per, rows_per)],
                            out_hbm.at[c, pl.ds(s * rows_per, rows_per)])
            plsc.subcore_barrier()

    if nph == 2:
        def kb(h0, h1, src_hbm, dst_hbm, z_hbm, o0, o1, isrc, idst, rows0,
               rows1, acc_sh, semg0, semg1, sems0, sems1):
            _impl([h0, h1], src_hbm, dst_hbm, z_hbm, [o0, o1], isrc, idst,
                  rows0, rows1, acc_sh, semg0, semg1, sems0, sems1)
    else:
        def kb(h0, src_hbm, dst_hbm, z_hbm, o0, isrc, idst, rows0,
               rows1, acc_sh, semg0, semg1, sems0, sems1):
            _impl([h0], src_hbm, dst_hbm, z_hbm, [o0], isrc, idst,
                  rows0, rows1, acc_sh, semg0, semg1, sems0, sems1)

    return k(kb)(*h_list, src2, dst2, zeros_slab)


def _tc_layer1(x_pad, W1, deg3, r_blk):
    np_, d_in = x_pad.shape
    d_h = W1.shape[1]
    half = d_h // 2
    grid = np_ // r_blk

    def body(x_ref, w_ref, deg_ref, lo_ref, hi_ref, dis_ref):
        deg = deg_ref[...]
        dis = lax.rsqrt(deg[0] + deg[1] + 1.0)
        h = jnp.dot(x_ref[...], w_ref[...], preferred_element_type=jnp.float32)
        lo_ref[...] = h[:, :half] * dis
        hi_ref[...] = h[:, half:] * dis
        dis_ref[...] = dis

    return pl.pallas_call(
        body,
        grid=(grid,),
        in_specs=[
            pl.BlockSpec((r_blk, d_in), lambda i: (i, 0)),
            pl.BlockSpec((d_in, d_h), lambda i: (0, 0)),
            pl.BlockSpec((_NC, r_blk, 1), lambda i: (0, i, 0)),
        ],
        out_specs=[
            pl.BlockSpec((r_blk, half), lambda i: (i, 0)),
            pl.BlockSpec((r_blk, half), lambda i: (i, 0)),
            pl.BlockSpec((r_blk, 1), lambda i: (i, 0)),
        ],
        out_shape=[
            jax.ShapeDtypeStruct((np_, half), jnp.float32),
            jax.ShapeDtypeStruct((np_, half), jnp.float32),
            jax.ShapeDtypeStruct((np_, 1), jnp.float32),
        ],
    )(x_pad, W1, deg3)


def _tc_layer2(acc_lo, acc_hi, h_lo, h_hi, dis, b1, W2, r_blk):
    np_ = h_lo.shape[0]
    half = h_lo.shape[1]
    d_out = W2.shape[1]
    grid = np_ // r_blk

    def body(alo_ref, ahi_ref, hlo_ref, hhi_ref, dis_ref, b1_ref, w2_ref,
             out_ref):
        dis_v = dis_ref[...]
        alo = alo_ref[...]
        ahi = ahi_ref[...]
        b1v = b1_ref[...]
        zlo = jnp.maximum((alo[0] + alo[1] + hlo_ref[...]) * dis_v
                          + b1v[:, :half], 0.0)
        zhi = jnp.maximum((ahi[0] + ahi[1] + hhi_ref[...]) * dis_v
                          + b1v[:, half:], 0.0)
        w2 = w2_ref[...]
        h2 = (jnp.dot(zlo, w2[:half], preferred_element_type=jnp.float32)
              + jnp.dot(zhi, w2[half:], preferred_element_type=jnp.float32))
        out_ref[...] = h2 * dis_v

    acc_spec = pl.BlockSpec((_NC, r_blk, half), lambda i: (0, i, 0))
    row_spec = pl.BlockSpec((r_blk, half), lambda i: (i, 0))
    return pl.pallas_call(
        body,
        grid=(grid,),
        in_specs=[
            acc_spec,
            acc_spec,
            row_spec,
            row_spec,
            pl.BlockSpec((r_blk, 1), lambda i: (i, 0)),
            pl.BlockSpec((1, 2 * half), lambda i: (0, 0)),
            pl.BlockSpec((2 * half, d_out), lambda i: (0, 0)),
        ],
        out_specs=pl.BlockSpec((r_blk, d_out), lambda i: (i, 0)),
        out_shape=jax.ShapeDtypeStruct((np_, d_out), jnp.float32),
    )(acc_lo, acc_hi, h_lo, h_hi, dis, b1, W2)


def _tc_final(acc2, h2, dis, b2, r_blk):
    np_, d_out = h2.shape
    grid = np_ // r_blk

    def body(a_ref, h_ref, dis_ref, b_ref, out_ref):
        a = a_ref[...]
        out_ref[...] = (a[0] + a[1] + h_ref[...]) * dis_ref[...] + b_ref[...]

    return pl.pallas_call(
        body,
        grid=(grid,),
        in_specs=[
            pl.BlockSpec((_NC, r_blk, d_out), lambda i: (0, i, 0)),
            pl.BlockSpec((r_blk, d_out), lambda i: (i, 0)),
            pl.BlockSpec((r_blk, 1), lambda i: (i, 0)),
            pl.BlockSpec((1, d_out), lambda i: (0, 0)),
        ],
        out_specs=pl.BlockSpec((r_blk, d_out), lambda i: (i, 0)),
        out_shape=jax.ShapeDtypeStruct((np_, d_out), jnp.float32),
    )(acc2, h2, dis, b2)


def kernel(x, edge_index, W1, b1, W2, b2):
    n, d_in = x.shape
    d_h = W1.shape[1]
    e = edge_index.shape[1]
    assert d_h % 256 == 0 and d_in % 128 == 0
    np_ = -(-n // 2048) * 2048
    r_blk = 1024
    nw = _NC * _NS

    niter = -(-e // (nw * _IB * _CHUNK)) * _IB
    e_pad = nw * niter * _CHUNK
    src = edge_index[0]
    dst = edge_index[1]
    pad_src = jnp.arange(e_pad - e, dtype=jnp.int32) % n
    src2 = jnp.concatenate([src, pad_src]).reshape(-1, _CHUNK)
    pad_dst = n + jnp.arange(e_pad - e, dtype=jnp.int32) % (np_ - n)
    dst2 = jnp.concatenate([dst, pad_dst]).reshape(-1, _CHUNK)
    zeros_flat = jnp.zeros((np_,), jnp.float32)
    zeros_slab = jnp.zeros((np_ // _NS, d_h // 2), jnp.float32)

    deg = _sc_degree(dst2, zeros_flat, np_)
    deg3 = deg.reshape(_NC, np_, 1)
    x_pad = jnp.pad(x, ((0, np_ - n), (0, 0)))
    h_lo, h_hi, dis = _tc_layer1(x_pad, W1, deg3, r_blk)
    acc_lo, acc_hi = _sc_scatter([h_lo, h_hi], src2, dst2, zeros_slab, np_)
    h2 = _tc_layer2(acc_lo, acc_hi, h_lo, h_hi, dis,
                    b1.reshape(1, -1), W2, r_blk)
    acc2, = _sc_scatter([h2], src2, dst2, zeros_slab, np_)
    out = _tc_final(acc2, h2, dis, b2.reshape(1, -1), r_blk)
    return out[:n]
